# baseline (speedup 1.0000x reference)
import jax
import jax.numpy as jnp
from jax import lax
from jax.experimental import pallas as pl

N_CHUNK = 512


def _body(nodes_ref, w_ref, out_ref):
    j = pl.program_id(0)
    nmax = pl.num_programs(0)

    w = w_ref[...]
    x = nodes_ref[...]
    part = lax.dot_general(
        w, x,
        dimension_numbers=(((1,), (1,)), ((0,), (0,))),
        preferred_element_type=jnp.float32,
    )

    @pl.when(j == 0)
    def _():
        out_ref[...] = jnp.zeros_like(out_ref)

    out_ref[...] += part[:, 0, :]

    @pl.when(j == nmax - 1)
    def _():
        out_ref[...] *= out_ref.dtype.type(1.0 / 4096.0)


def kernel(nodes, weights):
    B, N, F = nodes.shape
    grid = (N // N_CHUNK,)
    return pl.pallas_call(
        _body,
        grid=grid,
        in_specs=[
            pl.BlockSpec((B, N_CHUNK, F), lambda j: (0, j, 0)),
            pl.BlockSpec((B, N_CHUNK, 1), lambda j: (0, j, 0)),
        ],
        out_specs=pl.BlockSpec((B, F), lambda j: (0, 0)),
        out_shape=jax.ShapeDtypeStruct((B, F), jnp.float32),
    )(nodes, weights)


# speedup vs baseline: 1.0249x; 1.0249x over previous
import jax
import jax.numpy as jnp
from jax import lax
from jax.experimental import pallas as pl

N_CHUNK = 512


def _body(nodes_ref, w_ref, out_ref):
    j = pl.program_id(0)
    nmax = pl.num_programs(0)

    w = w_ref[...]
    x = nodes_ref[...]
    part = jnp.sum(x * w, axis=1)

    @pl.when(j == 0)
    def _():
        out_ref[...] = jnp.zeros_like(out_ref)

    out_ref[...] += part

    @pl.when(j == nmax - 1)
    def _():
        out_ref[...] *= out_ref.dtype.type(1.0 / 4096.0)


def kernel(nodes, weights):
    B, N, F = nodes.shape
    grid = (N // N_CHUNK,)
    return pl.pallas_call(
        _body,
        grid=grid,
        in_specs=[
            pl.BlockSpec((B, N_CHUNK, F), lambda j: (0, j, 0)),
            pl.BlockSpec((B, N_CHUNK, 1), lambda j: (0, j, 0)),
        ],
        out_specs=pl.BlockSpec((B, F), lambda j: (0, 0)),
        out_shape=jax.ShapeDtypeStruct((B, F), jnp.float32),
    )(nodes, weights)
